# Initial kernel scaffold; baseline (speedup 1.0000x reference)
import jax
import jax.numpy as jnp
from jax.experimental import pallas as pl


def kernel(input_ids, position_ids, word_embeddings, position_embeddings):
    raise NotImplementedError("write your pallas kernel here")


# speedup vs baseline: 1.2137x; 1.2137x over previous
import functools

import jax
import jax.numpy as jnp
from jax import lax
from jax.experimental import pallas as pl
from jax.experimental.pallas import tpu as pltpu
from jax.experimental.pallas import tpu_sc as plsc

B = 1024
S = 200
D = 64
N = B * S
VOCAB = 1000000
NPOS = 201

NC = 2
NS = 16
NW = NC * NS
PER_W = N // NW

G = 128
JSUB = 5
CHUNK = G * JSUB
NCHUNK = PER_W // CHUNK

_mesh = plsc.VectorSubcoreMesh(core_axis_name="c", subcore_axis_name="s")


@functools.partial(
    pl.kernel,
    out_type=jax.ShapeDtypeStruct((N, D), jnp.float32),
    mesh=_mesh,
    compiler_params=pltpu.CompilerParams(use_tc_tiling_on_sc=False),
    scratch_types=[
        pltpu.VMEM((PER_W // G, G), jnp.int32),
        pltpu.VMEM((PER_W // G, G), jnp.int32),
        pltpu.VMEM((CHUNK, D), jnp.float32),
        pltpu.VMEM((CHUNK, D), jnp.float32),
        pltpu.SemaphoreType.DMA,
    ],
)
def _emb(widx_hbm, pidx_hbm, wtab_hbm, ptab_hbm, out_hbm,
         widx_v, pidx_v, rows_v, prow_v, sem):
    cid = lax.axis_index("c")
    sid = lax.axis_index("s")
    wid = sid * NC + cid

    pltpu.sync_copy(widx_hbm.at[wid], widx_v)
    pltpu.sync_copy(pidx_hbm.at[wid], pidx_v)

    def chunk_body(c, _):
        copies = []
        for j in range(JSUB):
            copies.append(
                pltpu.async_copy(
                    wtab_hbm.at[widx_v.at[c * JSUB + j]],
                    rows_v.at[pl.ds(j * G, G)],
                    sem,
                )
            )
            copies.append(
                pltpu.async_copy(
                    ptab_hbm.at[pidx_v.at[c * JSUB + j]],
                    prow_v.at[pl.ds(j * G, G)],
                    sem,
                )
            )
        for cp in copies:
            cp.wait()

        def add_body(r, _):
            for v in range(D // 16):
                sl = pl.ds(v * 16, 16)
                plsc.addupdate(rows_v.at[r, sl], prow_v[r, sl])
            return 0

        lax.fori_loop(0, CHUNK, add_body, 0)

        pltpu.sync_copy(rows_v, out_hbm.at[pl.ds(wid * PER_W + c * CHUNK, CHUNK)])
        return 0

    lax.fori_loop(0, NCHUNK, chunk_body, 0)


def kernel(input_ids, position_ids, word_embeddings, position_embeddings):
    widx = input_ids.reshape(NW, PER_W // G, G)
    pidx = position_ids.reshape(NW, PER_W // G, G)
    out = _emb(widx, pidx, word_embeddings, position_embeddings)
    return out.reshape(B, S, D)


# speedup vs baseline: 1.2265x; 1.0105x over previous
import functools

import jax
import jax.numpy as jnp
from jax import lax
from jax.experimental import pallas as pl
from jax.experimental.pallas import tpu as pltpu
from jax.experimental.pallas import tpu_sc as plsc

B = 1024
S = 200
D = 64
N = B * S
VOCAB = 1000000
NPOS = 201

NC = 2
NS = 16
NW = NC * NS
PER_W = N // NW

G = 80
JSUB = 4
CHUNK = G * JSUB
NCHUNK = PER_W // CHUNK

_mesh = plsc.VectorSubcoreMesh(core_axis_name="c", subcore_axis_name="s")


@functools.partial(
    pl.kernel,
    out_type=jax.ShapeDtypeStruct((N, D), jnp.float32),
    mesh=_mesh,
    compiler_params=pltpu.CompilerParams(use_tc_tiling_on_sc=False),
    scratch_types=[
        pltpu.VMEM((PER_W // G, G), jnp.int32),
        pltpu.VMEM((PER_W // G, G), jnp.int32),
        pltpu.VMEM((CHUNK, D), jnp.float32),
        pltpu.VMEM((CHUNK, D), jnp.float32),
        pltpu.VMEM((CHUNK, D), jnp.float32),
        pltpu.VMEM((CHUNK, D), jnp.float32),
        pltpu.SemaphoreType.DMA,
        pltpu.SemaphoreType.DMA,
    ],
)
def _emb(widx_hbm, pidx_hbm, wtab_hbm, ptab_hbm, out_hbm,
         widx_v, pidx_v, rows0, rows1, prow0, prow1, sem0, sem1):
    cid = lax.axis_index("c")
    sid = lax.axis_index("s")
    wid = sid * NC + cid

    rows = (rows0, rows1)
    prow = (prow0, prow1)
    sems = (sem0, sem1)

    pltpu.sync_copy(widx_hbm.at[wid], widx_v)
    pltpu.sync_copy(pidx_hbm.at[wid], pidx_v)

    def fire(c, b):
        for j in range(JSUB):
            pltpu.async_copy(
                wtab_hbm.at[widx_v.at[c * JSUB + j]],
                rows[b].at[pl.ds(j * G, G)],
                sems[b],
            )
            pltpu.async_copy(
                ptab_hbm.at[pidx_v.at[c * JSUB + j]],
                prow[b].at[pl.ds(j * G, G)],
                sems[b],
            )

    def drain(b):
        for j in range(JSUB):
            pltpu.make_async_copy(
                wtab_hbm.at[widx_v.at[j]],
                rows[b].at[pl.ds(j * G, G)],
                sems[b],
            ).wait()
            pltpu.make_async_copy(
                ptab_hbm.at[pidx_v.at[j]],
                prow[b].at[pl.ds(j * G, G)],
                sems[b],
            ).wait()

    def add(b):
        def add_body(r, _):
            for rr in range(2):
                for v in range(D // 16):
                    sl = pl.ds(v * 16, 16)
                    plsc.addupdate(rows[b].at[2 * r + rr, sl], prow[b][2 * r + rr, sl])
            return 0

        lax.fori_loop(0, CHUNK // 2, add_body, 0)

    def writeback(c, b):
        pltpu.sync_copy(rows[b], out_hbm.at[pl.ds(wid * PER_W + c * CHUNK, CHUNK)])

    fire(0, 0)

    def loop_body(i, _):
        cbase = 2 * i
        for b in range(2):
            c = cbase + b
            fire(c + 1, 1 - b)
            drain(b)
            add(b)
            writeback(c, b)
        return 0

    lax.fori_loop(0, (NCHUNK - 2) // 2, loop_body, 0)

    fire(NCHUNK - 1, 1)
    drain(0)
    add(0)
    writeback(NCHUNK - 2, 0)
    drain(1)
    add(1)
    writeback(NCHUNK - 1, 1)


def kernel(input_ids, position_ids, word_embeddings, position_embeddings):
    widx = input_ids.reshape(NW, PER_W // G, G)
    pidx = position_ids.reshape(NW, PER_W // G, G)
    out = _emb(widx, pidx, word_embeddings, position_embeddings)
    return out.reshape(B, S, D)
